# baseline (speedup 1.0000x reference)
import functools

import jax
import jax.numpy as jnp
from jax import lax
from jax.experimental import pallas as pl
from jax.experimental.pallas import tpu as pltpu
from jax.experimental.pallas import tpu_sc as plsc

_B, _N, _D = 8, 50000, 128
_K = 200
_NC, _NS = 2, 16
_WPB = 4
_CHUNK = 64
_TAIL = _K - 3 * _CHUNK

_mesh = plsc.VectorSubcoreMesh(core_axis_name="c", subcore_axis_name="s")


@functools.partial(
    pl.kernel,
    mesh=_mesh,
    out_type=(
        jax.ShapeDtypeStruct((_B * _K, _D), jnp.float32),
        jax.ShapeDtypeStruct((_B * _K, _D), jnp.float32),
    ),
    scratch_types=[
        pltpu.VMEM((_B, _K), jnp.int32),
        pltpu.VMEM((_CHUNK,), jnp.int32),
        pltpu.VMEM((_CHUNK, _D), jnp.float32),
        pltpu.SemaphoreType.DMA,
    ],
)
def _sc_gather(table_hbm, idx_hbm, out_a, out_b, idx_all, idx_v, rows_v, sem):
    wid = lax.axis_index("s") * _NC + lax.axis_index("c")
    b = wid // _WPB
    w4 = wid % _WPB
    row_off = b * _N
    pltpu.sync_copy(idx_hbm, idx_all)

    @pl.when(w4 < _WPB - 1)
    def _full_chunk():
        koff = w4 * _CHUNK
        for j in range(_CHUNK // 16):
            idx_v[pl.ds(j * 16, 16)] = (
                idx_all[b, pl.ds(koff + j * 16, 16)] + row_off
            )
        pltpu.async_copy(table_hbm.at[idx_v], rows_v, sem).wait()
        base = b * _K + koff
        pltpu.sync_copy(rows_v, out_a.at[pl.ds(base, _CHUNK)])
        pltpu.sync_copy(rows_v, out_b.at[pl.ds(base, _CHUNK)])

    @pl.when(w4 == _WPB - 1)
    def _tail_chunk():
        idx_v[pl.ds(0, 16)] = idx_all[b, pl.ds(_K - 16, 16)] + row_off
        pltpu.async_copy(
            table_hbm.at[idx_v.at[pl.ds(0, 16)]], rows_v.at[pl.ds(0, 16)], sem
        ).wait()
        out_base = b * _K + _K - _TAIL
        src = rows_v.at[pl.ds(16 - _TAIL, _TAIL)]
        pltpu.sync_copy(src, out_a.at[pl.ds(out_base, _TAIL)])
        pltpu.sync_copy(src, out_b.at[pl.ds(out_base, _TAIL)])


def kernel(feature, tail_id):
    table = feature.reshape(_B * _N, _D)
    out_a, out_b = _sc_gather(table, tail_id)
    shape = (_B, _K, _D)
    return (out_a.reshape(shape), out_b.reshape(shape))


# speedup vs baseline: 1.0572x; 1.0572x over previous
import functools

import jax
import jax.numpy as jnp
from jax import lax
from jax.experimental import pallas as pl
from jax.experimental.pallas import tpu as pltpu
from jax.experimental.pallas import tpu_sc as plsc

_B, _N, _D = 8, 50000, 128
_K = 200
_NC, _NS = 2, 16
_WPB = 4
_CHUNK = 64
_TAIL = _K - 3 * _CHUNK

_mesh = plsc.VectorSubcoreMesh(core_axis_name="c", subcore_axis_name="s")


@functools.partial(
    pl.kernel,
    mesh=_mesh,
    out_type=(
        jax.ShapeDtypeStruct((_B * _K, _D), jnp.float32),
        jax.ShapeDtypeStruct((_B * _K, _D), jnp.float32),
    ),
    scratch_types=[
        pltpu.VMEM((_CHUNK,), jnp.int32),
        pltpu.VMEM((_CHUNK, _D), jnp.float32),
        pltpu.SemaphoreType.DMA,
    ],
)
def _sc_gather(table_hbm, idx_hbm, out_a, out_b, idx_v, rows_v, sem):
    wid = lax.axis_index("s") * _NC + lax.axis_index("c")
    b = wid // _WPB
    w4 = wid % _WPB
    row_off = b * _N

    @pl.when(w4 < _WPB - 1)
    def _full_chunk():
        base = b * _K + w4 * _CHUNK
        pltpu.sync_copy(idx_hbm.at[pl.ds(base, _CHUNK)], idx_v)
        for j in range(_CHUNK // 16):
            sl = pl.ds(j * 16, 16)
            idx_v[sl] = idx_v[sl] + row_off
        pltpu.async_copy(table_hbm.at[idx_v], rows_v, sem).wait()
        cp_a = pltpu.async_copy(rows_v, out_a.at[pl.ds(base, _CHUNK)], sem)
        cp_b = pltpu.async_copy(rows_v, out_b.at[pl.ds(base, _CHUNK)], sem)
        cp_a.wait()
        cp_b.wait()

    @pl.when(w4 == _WPB - 1)
    def _tail_chunk():
        base = b * _K + _K - 16
        pltpu.sync_copy(idx_hbm.at[pl.ds(base, 16)], idx_v.at[pl.ds(0, 16)])
        idx_v[pl.ds(0, 16)] = idx_v[pl.ds(0, 16)] + row_off
        pltpu.async_copy(
            table_hbm.at[idx_v.at[pl.ds(0, 16)]], rows_v.at[pl.ds(0, 16)], sem
        ).wait()
        out_base = b * _K + _K - _TAIL
        src = rows_v.at[pl.ds(16 - _TAIL, _TAIL)]
        cp_a = pltpu.async_copy(src, out_a.at[pl.ds(out_base, _TAIL)], sem)
        cp_b = pltpu.async_copy(src, out_b.at[pl.ds(out_base, _TAIL)], sem)
        cp_a.wait()
        cp_b.wait()


def kernel(feature, tail_id):
    table = feature.reshape(_B * _N, _D)
    out_a, out_b = _sc_gather(table, tail_id.reshape(_B * _K))
    shape = (_B, _K, _D)
    return (out_a.reshape(shape), out_b.reshape(shape))


# speedup vs baseline: 1.0587x; 1.0013x over previous
import functools

import jax
import jax.numpy as jnp
from jax import lax
from jax.experimental import pallas as pl
from jax.experimental.pallas import tpu as pltpu
from jax.experimental.pallas import tpu_sc as plsc

_B, _N, _D = 8, 50000, 128
_K = 200
_NC, _NS = 2, 16
_WPB = 4
_CHUNK = 64
_LAST = _K - _CHUNK

_mesh = plsc.VectorSubcoreMesh(core_axis_name="c", subcore_axis_name="s")


@functools.partial(
    pl.kernel,
    mesh=_mesh,
    out_type=(
        jax.ShapeDtypeStruct((_B * _K, _D), jnp.float32),
        jax.ShapeDtypeStruct((_B * _K, _D), jnp.float32),
    ),
    scratch_types=[
        pltpu.VMEM((_CHUNK,), jnp.int32),
        pltpu.VMEM((_CHUNK, _D), jnp.float32),
        pltpu.SemaphoreType.DMA,
    ],
)
def _sc_gather(table_hbm, idx_hbm, out_a, out_b, idx_v, rows_v, sem):
    wid = lax.axis_index("s") * _NC + lax.axis_index("c")
    b = wid // _WPB
    w4 = wid % _WPB
    base = b * _K + jnp.minimum(w4 * _CHUNK, _LAST)
    pltpu.sync_copy(idx_hbm.at[pl.ds(base, _CHUNK)], idx_v)
    row_off = b * _N
    for j in range(_CHUNK // 16):
        sl = pl.ds(j * 16, 16)
        idx_v[sl] = idx_v[sl] + row_off
    pltpu.async_copy(table_hbm.at[idx_v], rows_v, sem).wait()
    cp_a = pltpu.async_copy(rows_v, out_a.at[pl.ds(base, _CHUNK)], sem)
    cp_b = pltpu.async_copy(rows_v, out_b.at[pl.ds(base, _CHUNK)], sem)
    cp_a.wait()
    cp_b.wait()


def kernel(feature, tail_id):
    table = feature.reshape(_B * _N, _D)
    out_a, out_b = _sc_gather(table, tail_id.reshape(_B * _K))
    shape = (_B, _K, _D)
    return (out_a.reshape(shape), out_b.reshape(shape))
